# baseline (speedup 1.0000x reference)
import functools

import jax
import jax.numpy as jnp
from jax import lax
from jax.experimental import pallas as pl
from jax.experimental.pallas import tpu as pltpu
from jax.experimental.pallas import tpu_sc as plsc

N = 10000
NP = 10240
D = 128
E = 320000
ALPHA = 0.1
ITERS = 10

NSUB = 16
CHUNK = 128
EPT_REAL = E // NSUB
NCH = 158
EPT = NCH * CHUNK
EPT_ALLOC = EPT + CHUNK
ROWS_PT = NP // NSUB
RCH = 128


def _matmul(x, W):
    def body(x_ref, w_ref, o_ref):
        o_ref[...] = jnp.dot(x_ref[...], w_ref[...],
                             preferred_element_type=jnp.float32)

    return pl.pallas_call(
        body,
        grid=(10,),
        in_specs=[
            pl.BlockSpec((NP // 10, D), lambda i: (i, 0)),
            pl.BlockSpec((D, D), lambda i: (0, 0)),
        ],
        out_specs=pl.BlockSpec((NP // 10, D), lambda i: (i, 0)),
        out_shape=jax.ShapeDtypeStruct((NP, D), jnp.float32),
    )(x, W)


def _pad_edges(a):
    a = a.reshape(NSUB, EPT_REAL)
    a = jnp.pad(a, ((0, 0), (0, EPT_ALLOC - EPT_REAL)))
    return a.reshape(-1)


def _sc_propagate(x0, srcp, dstp, valp):
    mesh = plsc.VectorSubcoreMesh(
        core_axis_name="c", subcore_axis_name="s", num_cores=1)

    @functools.partial(
        pl.kernel,
        out_type=jax.ShapeDtypeStruct((NP, D), jnp.float32),
        mesh=mesh,
        compiler_params=pltpu.CompilerParams(needs_layout_passes=False),
        scratch_types=[
            pltpu.VMEM_SHARED((NP, D), jnp.float32),
            pltpu.VMEM((CHUNK, D), jnp.float32),
            pltpu.VMEM((CHUNK, D), jnp.float32),
            pltpu.VMEM((CHUNK,), jnp.int32),
            pltpu.VMEM((CHUNK,), jnp.int32),
            pltpu.VMEM((CHUNK,), jnp.int32),
            pltpu.VMEM((CHUNK,), jnp.int32),
            pltpu.VMEM((CHUNK,), jnp.float32),
            pltpu.VMEM((CHUNK,), jnp.float32),
            pltpu.SemaphoreType.DMA,
            pltpu.SemaphoreType.DMA,
        ],
    )
    def prop(x0_h, src_h, dst_h, val_h, out_h, agg, rowsA, rowsB,
             sidxA, sidxB, didxA, didxB, valA, valB, semA, semB):
        sid = lax.axis_index("s")
        ebase = sid * EPT_ALLOC
        row0 = sid * ROWS_PT

        def load_idx(g, sidx, didx, val):
            return
            off = ebase + g * CHUNK
            pltpu.sync_copy(src_h.at[pl.ds(off, CHUNK)], sidx)
            pltpu.sync_copy(dst_h.at[pl.ds(off, CHUNK)], didx)
            pltpu.sync_copy(val_h.at[pl.ds(off, CHUNK)], val)

        def zidx(buf):
            for d in range(CHUNK // 16):
                buf[pl.ds(d * 16, 16)] = jnp.zeros((16,), jnp.int32)
        zidx(sidxA)
        zidxB = zidx(sidxB)

        def scale(rows, val):
            return
            def body(e, carry):
                vv = plsc.load_gather(val, [jnp.full((16,), e, jnp.int32)])
                for d in range(D // 16):
                    sl = pl.ds(d * 16, 16)
                    rows[e, sl] = rows[e, sl] * vv
                return carry
            lax.fori_loop(0, CHUNK, body, 0, unroll=8)

        for j in range(ROWS_PT // RCH):
            r = row0 + j * RCH
            pltpu.sync_copy(x0_h.at[pl.ds(r, RCH)], rowsA.at[pl.ds(0, RCH)])
            pltpu.sync_copy(rowsA.at[pl.ds(0, RCH)], out_h.at[pl.ds(r, RCH)])
        plsc.subcore_barrier()

        def iter_body(it, carry):
            def zb(i, carry2):
                for d in range(D // 16):
                    rowsB[i, pl.ds(d * 16, 16)] = jnp.zeros((16,), jnp.float32)
                return carry2
            lax.fori_loop(0, RCH, zb, 0, unroll=8)
            for j in range(ROWS_PT // RCH):
                r = row0 + j * RCH
                pltpu.sync_copy(rowsB, agg.at[pl.ds(r, RCH)])
            plsc.subcore_barrier()

            load_idx(0, sidxA, didxA, valA)
            pltpu.make_async_copy(out_h.at[sidxA], rowsA, semA).start()

            def pair(p, c2):
                g = 2 * p
                load_idx(g + 1, sidxB, didxB, valB)
                pltpu.make_async_copy(out_h.at[sidxB], rowsB, semB).start()
                pltpu.make_async_copy(out_h.at[sidxA], rowsA, semA).wait()
                scale(rowsA, valA)
                load_idx(g + 2, sidxA, didxA, valA)

                @pl.when(p < NCH // 2 - 1)
                def _():
                    pltpu.make_async_copy(out_h.at[sidxA], rowsA, semA).start()

                pltpu.make_async_copy(out_h.at[sidxB], rowsB, semB).wait()
                scale(rowsB, valB)
                return c2

            lax.fori_loop(0, NCH // 2, pair, 0)
            plsc.subcore_barrier()

            for j in range(ROWS_PT // RCH):
                r = row0 + j * RCH
                pltpu.sync_copy(agg.at[pl.ds(r, RCH)], rowsA.at[pl.ds(0, RCH)])
                pltpu.sync_copy(x0_h.at[pl.ds(r, RCH)], rowsB.at[pl.ds(0, RCH)])

                def mix(i, c3):
                    for d in range(D // 16):
                        sl = pl.ds(d * 16, 16)
                        a = rowsA[i, sl]
                        p0 = rowsB[i, sl]
                        rowsA[i, sl] = jnp.maximum(
                            a * (1.0 - ALPHA) + p0 * ALPHA, 0.0)
                    return c3
                lax.fori_loop(0, RCH, mix, 0, unroll=4)
                pltpu.sync_copy(rowsA.at[pl.ds(0, RCH)], out_h.at[pl.ds(r, RCH)])
            plsc.subcore_barrier()
            return carry

        lax.fori_loop(0, ITERS, iter_body, 0)

    return prop(x0, srcp, dstp, valp)


def kernel(x, W, edge_index, edge_vals):
    xp = jnp.pad(x, ((0, NP - N), (0, 0)))
    x0 = _matmul(xp, W)
    dst = edge_index[0]
    src = edge_index[1]
    srcp = _pad_edges(src)
    dstp = _pad_edges(dst)
    valp = _pad_edges(edge_vals)
    return _sc_propagate(x0, srcp, dstp, valp)[:N]


# speedup vs baseline: 31.3753x; 31.3753x over previous
import functools

import jax
import jax.numpy as jnp
from jax import lax
from jax.experimental import pallas as pl
from jax.experimental.pallas import tpu as pltpu
from jax.experimental.pallas import tpu_sc as plsc

N = 10000
NP = 10240
D = 128
E = 320000
ALPHA = 0.1
ITERS = 10

NSUB = 16
CHUNK = 128
SCH = 8
NSC = 20
NCH = NSC * SCH
EPT = NCH * CHUNK
EPT_REAL = E // NSUB
ROWS_PT = NP // NSUB
RCH = 128
GBYTES = CHUNK * D * 4


def _matmul(x, W):
    def body(x_ref, w_ref, o_ref):
        o_ref[...] = jnp.dot(x_ref[...], w_ref[...],
                             preferred_element_type=jnp.float32)

    return pl.pallas_call(
        body,
        grid=(10,),
        in_specs=[
            pl.BlockSpec((NP // 10, D), lambda i: (i, 0)),
            pl.BlockSpec((D, D), lambda i: (0, 0)),
        ],
        out_specs=pl.BlockSpec((NP // 10, D), lambda i: (i, 0)),
        out_shape=jax.ShapeDtypeStruct((NP, D), jnp.float32),
    )(x, W)


def _pad_edges(a, fill):
    a = a.reshape(NSUB, EPT_REAL)
    pad = jnp.broadcast_to(fill[None, :], (NSUB, EPT - EPT_REAL))
    a = jnp.concatenate([a, pad.astype(a.dtype)], axis=1)
    return a.reshape(NSUB * NCH, CHUNK)


def _sc_propagate(x0, srcp, dstp, valp):
    mesh = plsc.VectorSubcoreMesh(
        core_axis_name="c", subcore_axis_name="s", num_cores=1)

    @functools.partial(
        pl.kernel,
        out_type=jax.ShapeDtypeStruct((NP, D), jnp.float32),
        mesh=mesh,
        compiler_params=pltpu.CompilerParams(needs_layout_passes=False),
        scratch_types=[
            pltpu.VMEM_SHARED((NP, D), jnp.float32),
            pltpu.VMEM((CHUNK, D), jnp.float32),
            pltpu.VMEM((CHUNK, D), jnp.float32),
            pltpu.VMEM((SCH, CHUNK), jnp.int32),
            pltpu.VMEM((SCH, CHUNK), jnp.int32),
            pltpu.VMEM((SCH, CHUNK), jnp.float32),
            pltpu.VMEM((SCH, CHUNK), jnp.int32),
            pltpu.VMEM((SCH, CHUNK), jnp.int32),
            pltpu.VMEM((SCH, CHUNK), jnp.float32),
            pltpu.VMEM((CHUNK,), jnp.int32),
            pltpu.VMEM((CHUNK,), jnp.int32),
            pltpu.VMEM((CHUNK,), jnp.float32),
            pltpu.SemaphoreType.DMA,
            pltpu.SemaphoreType.DMA,
            pltpu.SemaphoreType.DMA,
            pltpu.SemaphoreType.DMA,
            pltpu.SemaphoreType.DMA,
            pltpu.SemaphoreType.DMA,
        ],
    )
    def prop(x0_h, src_h, dst_h, val_h, out_h, agg, rowsA, rowsB,
             sid0, did0, val0, sid1, did1, val1, didxA, didxB, val1d,
             semGA, semGB, semSA, semSB, semI0, semI1):
        sid = lax.axis_index("s")
        crow0 = sid * NCH
        row0 = sid * ROWS_PT

        def idx_load(sc, sbuf, dbuf, vbuf, sem):
            r = crow0 + sc * SCH
            pltpu.make_async_copy(src_h.at[pl.ds(r, SCH)], sbuf, sem).start()
            pltpu.make_async_copy(dst_h.at[pl.ds(r, SCH)], dbuf, sem).start()
            pltpu.make_async_copy(val_h.at[pl.ds(r, SCH)], vbuf, sem).start()

        def idx_wait(sbuf, dbuf, vbuf, sem):
            pltpu.make_async_copy(src_h.at[pl.ds(0, SCH)], sbuf, sem).wait()
            pltpu.make_async_copy(dst_h.at[pl.ds(0, SCH)], dbuf, sem).wait()
            pltpu.make_async_copy(val_h.at[pl.ds(0, SCH)], vbuf, sem).wait()

        def g_start(sbuf, r, rows, sem):
            pltpu.make_async_copy(out_h.at[sbuf.at[r]], rows, sem).start()

        def g_wait(sbuf, r, rows, sem):
            pltpu.make_async_copy(out_h.at[sbuf.at[r]], rows, sem).wait()

        def sc_start(rows, dbuf, r, didx, sem):
            for d in range(CHUNK // 16):
                sl = pl.ds(d * 16, 16)
                didx[sl] = dbuf[r, sl]
            pltpu.make_async_copy(rows, agg.at[didx], sem).start(add=True)

        def sc_wait(sem):
            pltpu.make_async_copy(rowsA, agg.at[didxA], sem).wait()

        def scale(rows, vbuf, r):
            for d in range(CHUNK // 16):
                sl = pl.ds(d * 16, 16)
                val1d[sl] = vbuf[r, sl]

            def body(e, c):
                vv = plsc.load_gather(val1d, [jnp.full((16,), e, jnp.int32)])
                for d in range(D // 16):
                    sl = pl.ds(d * 16, 16)
                    rows[e, sl] = rows[e, sl] * vv
                return c
            lax.fori_loop(0, CHUNK, body, 0, unroll=8)

        def pipeline(sbuf, dbuf, vbuf, reload_fn, not_first=None):
            if not_first is None:
                sc_wait(semSA)
                sc_wait(semSB)
            else:
                @pl.when(not_first)
                def _():
                    sc_wait(semSA)
                    sc_wait(semSB)
            g_start(sbuf, 0, rowsA, semGA)
            g_start(sbuf, 1, rowsB, semGB)
            reload_fn()
            for p in range(SCH // 2):
                ra, rb = 2 * p, 2 * p + 1
                g_wait(sbuf, ra, rowsA, semGA)
                scale(rowsA, vbuf, ra)
                sc_start(rowsA, dbuf, ra, didxA, semSA)
                g_wait(sbuf, rb, rowsB, semGB)
                scale(rowsB, vbuf, rb)
                sc_start(rowsB, dbuf, rb, didxB, semSB)
                if p < SCH // 2 - 1:
                    sc_wait(semSA)
                    g_start(sbuf, ra + 2, rowsA, semGA)
                    sc_wait(semSB)
                    g_start(sbuf, rb + 2, rowsB, semGB)

        def zrows(buf):
            def zb(i, c):
                for d in range(D // 16):
                    buf[i, pl.ds(d * 16, 16)] = jnp.zeros((16,), jnp.float32)
                return c
            lax.fori_loop(0, RCH, zb, 0, unroll=8)

        zrows(rowsB)
        for j in range(ROWS_PT // RCH):
            r = row0 + j * RCH
            pltpu.sync_copy(x0_h.at[pl.ds(r, RCH)], rowsA)
            pltpu.sync_copy(rowsA, out_h.at[pl.ds(r, RCH)])
            pltpu.sync_copy(rowsB, agg.at[pl.ds(r, RCH)])
        plsc.subcore_barrier()

        def iter_body(it, carry):
            idx_load(0, sid0, did0, val0, semI0)

            def sp_body(sp, c2):
                idx_wait(sid0, did0, val0, semI0)

                def reload1():
                    idx_load(2 * sp + 1, sid1, did1, val1, semI1)
                pipeline(sid0, did0, val0, reload1, not_first=sp > 0)

                idx_wait(sid1, did1, val1, semI1)

                def reload0():
                    @pl.when(sp < NSC // 2 - 1)
                    def _():
                        idx_load(2 * sp + 2, sid0, did0, val0, semI0)
                pipeline(sid1, did1, val1, reload0)
                return c2

            lax.fori_loop(0, NSC // 2, sp_body, 0)
            sc_wait(semSA)
            sc_wait(semSB)
            plsc.subcore_barrier()

            for j in range(ROWS_PT // RCH):
                r = row0 + j * RCH
                pltpu.sync_copy(agg.at[pl.ds(r, RCH)], rowsA)
                pltpu.sync_copy(x0_h.at[pl.ds(r, RCH)], rowsB)

                def mix(i, c3):
                    for d in range(D // 16):
                        sl = pl.ds(d * 16, 16)
                        a = rowsA[i, sl]
                        p0 = rowsB[i, sl]
                        rowsB[i, sl] = jnp.maximum(
                            a * (1.0 - ALPHA) + p0 * ALPHA, 0.0)
                    return c3
                lax.fori_loop(0, RCH, mix, 0, unroll=4)
                zrows(rowsA)
                pltpu.sync_copy(rowsA, agg.at[pl.ds(r, RCH)])
                pltpu.sync_copy(rowsB, out_h.at[pl.ds(r, RCH)])
            plsc.subcore_barrier()
            return carry

        lax.fori_loop(0, ITERS, iter_body, 0)

    return prop(x0, srcp, dstp, valp)


def kernel(x, W, edge_index, edge_vals):
    xp = jnp.pad(x, ((0, NP - N), (0, 0)))
    x0 = _matmul(xp, W)
    dst = edge_index[0]
    src = edge_index[1]
    spread = (jnp.arange(EPT - EPT_REAL, dtype=jnp.int32) * 37) % N
    srcp = _pad_edges(src, spread)
    dstp = _pad_edges(dst, spread)
    valp = _pad_edges(edge_vals, jnp.zeros((EPT - EPT_REAL,), jnp.float32))
    return _sc_propagate(x0, srcp, dstp, valp)[:N]


# speedup vs baseline: 36.8672x; 1.1750x over previous
import functools

import jax
import jax.numpy as jnp
from jax import lax
from jax.experimental import pallas as pl
from jax.experimental.pallas import tpu as pltpu
from jax.experimental.pallas import tpu_sc as plsc

N = 10000
NP = 10240
D = 128
E = 320000
ALPHA = 0.1
ITERS = 10

NSUB = 16
CHUNK = 128
SCH = 8
NSC = 20
NCH = NSC * SCH
EPT = NCH * CHUNK
EPT_REAL = E // NSUB
ROWS_PT = NP // NSUB
RCH = 128
GBYTES = CHUNK * D * 4


def _matmul(x, W):
    def body(x_ref, w_ref, o_ref):
        o_ref[...] = jnp.dot(x_ref[...], w_ref[...],
                             preferred_element_type=jnp.float32)

    return pl.pallas_call(
        body,
        grid=(10,),
        in_specs=[
            pl.BlockSpec((NP // 10, D), lambda i: (i, 0)),
            pl.BlockSpec((D, D), lambda i: (0, 0)),
        ],
        out_specs=pl.BlockSpec((NP // 10, D), lambda i: (i, 0)),
        out_shape=jax.ShapeDtypeStruct((NP, D), jnp.float32),
    )(x, W)


def _pad_edges(a, fill):
    a = a.reshape(NSUB, EPT_REAL)
    pad = jnp.broadcast_to(fill[None, :], (NSUB, EPT - EPT_REAL))
    a = jnp.concatenate([a, pad.astype(a.dtype)], axis=1)
    return a.reshape(NSUB * NCH, CHUNK)


def _sc_propagate(x0, srcp, dstp, valp):
    mesh = plsc.VectorSubcoreMesh(
        core_axis_name="c", subcore_axis_name="s", num_cores=1)

    @functools.partial(
        pl.kernel,
        out_type=jax.ShapeDtypeStruct((NP, D), jnp.float32),
        mesh=mesh,
        compiler_params=pltpu.CompilerParams(needs_layout_passes=False),
        scratch_types=[
            pltpu.VMEM_SHARED((NP, D), jnp.float32),
            pltpu.VMEM((CHUNK, D), jnp.float32),
            pltpu.VMEM((CHUNK, D), jnp.float32),
            pltpu.VMEM((SCH, CHUNK), jnp.int32),
            pltpu.VMEM((SCH, CHUNK), jnp.int32),
            pltpu.VMEM((SCH, CHUNK), jnp.float32),
            pltpu.VMEM((SCH, CHUNK), jnp.int32),
            pltpu.VMEM((SCH, CHUNK), jnp.int32),
            pltpu.VMEM((SCH, CHUNK), jnp.float32),
            pltpu.VMEM((CHUNK,), jnp.int32),
            pltpu.VMEM((CHUNK,), jnp.int32),
            pltpu.VMEM((CHUNK,), jnp.float32),
            pltpu.SemaphoreType.DMA,
            pltpu.SemaphoreType.DMA,
            pltpu.SemaphoreType.DMA,
            pltpu.SemaphoreType.DMA,
            pltpu.SemaphoreType.DMA,
            pltpu.SemaphoreType.DMA,
        ],
    )
    def prop(x0_h, src_h, dst_h, val_h, out_h, agg, rowsA, rowsB,
             sid0, did0, val0, sid1, did1, val1, didxA, didxB, val1d,
             semGA, semGB, semSA, semSB, semI0, semI1):
        sid = lax.axis_index("s")
        crow0 = sid * NCH
        row0 = sid * ROWS_PT

        def idx_load(sc, sbuf, dbuf, vbuf, sem):
            r = crow0 + sc * SCH
            pltpu.make_async_copy(src_h.at[pl.ds(r, SCH)], sbuf, sem).start()
            pltpu.make_async_copy(dst_h.at[pl.ds(r, SCH)], dbuf, sem).start()
            pltpu.make_async_copy(val_h.at[pl.ds(r, SCH)], vbuf, sem).start()

        def idx_wait(sbuf, dbuf, vbuf, sem):
            pltpu.make_async_copy(src_h.at[pl.ds(0, SCH)], sbuf, sem).wait()
            pltpu.make_async_copy(dst_h.at[pl.ds(0, SCH)], dbuf, sem).wait()
            pltpu.make_async_copy(val_h.at[pl.ds(0, SCH)], vbuf, sem).wait()

        def g_start(sbuf, r, rows, sem):
            pltpu.make_async_copy(out_h.at[sbuf.at[r]], rows, sem).start()

        def g_wait(sbuf, r, rows, sem):
            pltpu.make_async_copy(out_h.at[sbuf.at[r]], rows, sem).wait()

        def sc_start(rows, dbuf, r, didx, sem):
            for d in range(CHUNK // 16):
                sl = pl.ds(d * 16, 16)
                didx[sl] = dbuf[r, sl]
            pltpu.make_async_copy(rows, agg.at[didx], sem).start(add=True)

        def sc_wait(sem):
            pltpu.make_async_copy(rowsA, agg.at[didxA], sem).wait()

        def scale(rows, vbuf, r):
            for d in range(CHUNK // 16):
                sl = pl.ds(d * 16, 16)
                val1d[sl] = vbuf[r, sl]

            @plsc.parallel_loop(0, CHUNK, 1, unroll=8)
            def _(e):
                vv = plsc.load_gather(val1d, [jnp.full((16,), e, jnp.int32)])
                for d in range(D // 16):
                    sl = pl.ds(d * 16, 16)
                    rows[e, sl] = rows[e, sl] * vv

        def pipeline(sbuf, dbuf, vbuf, reload_fn, not_first=None):
            if not_first is None:
                sc_wait(semSA)
                sc_wait(semSB)
            else:
                @pl.when(not_first)
                def _():
                    sc_wait(semSA)
                    sc_wait(semSB)
            g_start(sbuf, 0, rowsA, semGA)
            g_start(sbuf, 1, rowsB, semGB)
            reload_fn()
            for p in range(SCH // 2):
                ra, rb = 2 * p, 2 * p + 1
                g_wait(sbuf, ra, rowsA, semGA)
                scale(rowsA, vbuf, ra)
                sc_start(rowsA, dbuf, ra, didxA, semSA)
                g_wait(sbuf, rb, rowsB, semGB)
                scale(rowsB, vbuf, rb)
                sc_start(rowsB, dbuf, rb, didxB, semSB)
                if p < SCH // 2 - 1:
                    sc_wait(semSA)
                    g_start(sbuf, ra + 2, rowsA, semGA)
                    sc_wait(semSB)
                    g_start(sbuf, rb + 2, rowsB, semGB)

        def zrows(buf):
            def zb(i, c):
                for d in range(D // 16):
                    buf[i, pl.ds(d * 16, 16)] = jnp.zeros((16,), jnp.float32)
                return c
            lax.fori_loop(0, RCH, zb, 0, unroll=8)

        zrows(rowsB)
        for j in range(ROWS_PT // RCH):
            r = row0 + j * RCH
            pltpu.sync_copy(x0_h.at[pl.ds(r, RCH)], rowsA)
            pltpu.sync_copy(rowsA, out_h.at[pl.ds(r, RCH)])
            pltpu.sync_copy(rowsB, agg.at[pl.ds(r, RCH)])
        plsc.subcore_barrier()

        def iter_body(it, carry):
            idx_load(0, sid0, did0, val0, semI0)

            def sp_body(sp, c2):
                idx_wait(sid0, did0, val0, semI0)

                def reload1():
                    idx_load(2 * sp + 1, sid1, did1, val1, semI1)
                pipeline(sid0, did0, val0, reload1, not_first=sp > 0)

                idx_wait(sid1, did1, val1, semI1)

                def reload0():
                    @pl.when(sp < NSC // 2 - 1)
                    def _():
                        idx_load(2 * sp + 2, sid0, did0, val0, semI0)
                pipeline(sid1, did1, val1, reload0)
                return c2

            lax.fori_loop(0, NSC // 2, sp_body, 0)
            sc_wait(semSA)
            sc_wait(semSB)
            plsc.subcore_barrier()

            for j in range(ROWS_PT // RCH):
                r = row0 + j * RCH
                pltpu.sync_copy(agg.at[pl.ds(r, RCH)], rowsA)
                pltpu.sync_copy(x0_h.at[pl.ds(r, RCH)], rowsB)

                def mix(i, c3):
                    for d in range(D // 16):
                        sl = pl.ds(d * 16, 16)
                        a = rowsA[i, sl]
                        p0 = rowsB[i, sl]
                        rowsB[i, sl] = jnp.maximum(
                            a * (1.0 - ALPHA) + p0 * ALPHA, 0.0)
                    return c3
                lax.fori_loop(0, RCH, mix, 0, unroll=4)
                zrows(rowsA)
                pltpu.sync_copy(rowsA, agg.at[pl.ds(r, RCH)])
                pltpu.sync_copy(rowsB, out_h.at[pl.ds(r, RCH)])
            plsc.subcore_barrier()
            return carry

        lax.fori_loop(0, ITERS, iter_body, 0)

    return prop(x0, srcp, dstp, valp)


def kernel(x, W, edge_index, edge_vals):
    xp = jnp.pad(x, ((0, NP - N), (0, 0)))
    x0 = _matmul(xp, W)
    dst = edge_index[0]
    src = edge_index[1]
    spread = (jnp.arange(EPT - EPT_REAL, dtype=jnp.int32) * 37) % N
    srcp = _pad_edges(src, spread)
    dstp = _pad_edges(dst, spread)
    valp = _pad_edges(edge_vals, jnp.zeros((EPT - EPT_REAL,), jnp.float32))
    return _sc_propagate(x0, srcp, dstp, valp)[:N]


# speedup vs baseline: 38.7006x; 1.0497x over previous
import functools

import jax
import jax.numpy as jnp
from jax import lax
from jax.experimental import pallas as pl
from jax.experimental.pallas import tpu as pltpu
from jax.experimental.pallas import tpu_sc as plsc

N = 10000
NP = 10240
D = 128
E = 320000
ALPHA = 0.1
ITERS = 10

NSUB = 16
CHUNK = 128
SCH = 8
NSC = 20
NCH = NSC * SCH
EPT = NCH * CHUNK
EPT_REAL = E // NSUB
ROWS_PT = NP // NSUB
RCH = 128
GBYTES = CHUNK * D * 4


def _matmul(x, W):
    def body(x_ref, w_ref, o_ref):
        o_ref[...] = jnp.dot(x_ref[...], w_ref[...],
                             preferred_element_type=jnp.float32)

    return pl.pallas_call(
        body,
        grid=(10,),
        in_specs=[
            pl.BlockSpec((NP // 10, D), lambda i: (i, 0)),
            pl.BlockSpec((D, D), lambda i: (0, 0)),
        ],
        out_specs=pl.BlockSpec((NP // 10, D), lambda i: (i, 0)),
        out_shape=jax.ShapeDtypeStruct((NP, D), jnp.float32),
    )(x, W)


def _pad_edges(a, fill):
    a = a.reshape(NSUB, EPT_REAL)
    pad = jnp.broadcast_to(fill[None, :], (NSUB, EPT - EPT_REAL))
    a = jnp.concatenate([a, pad.astype(a.dtype)], axis=1)
    return a.reshape(NSUB * NCH, CHUNK)


def _sc_propagate(x0, srcp, dstp, valp):
    mesh = plsc.VectorSubcoreMesh(
        core_axis_name="c", subcore_axis_name="s", num_cores=1)

    @functools.partial(
        pl.kernel,
        out_type=jax.ShapeDtypeStruct((NP, D), jnp.float32),
        mesh=mesh,
        compiler_params=pltpu.CompilerParams(needs_layout_passes=False),
        scratch_types=[
            pltpu.VMEM_SHARED((NP, D), jnp.float32),
            pltpu.VMEM((CHUNK, D), jnp.float32),
            pltpu.VMEM((CHUNK, D), jnp.float32),
            pltpu.VMEM((SCH, CHUNK), jnp.int32),
            pltpu.VMEM((SCH, CHUNK), jnp.int32),
            pltpu.VMEM((SCH, CHUNK), jnp.float32),
            pltpu.VMEM((SCH, CHUNK), jnp.int32),
            pltpu.VMEM((SCH, CHUNK), jnp.int32),
            pltpu.VMEM((SCH, CHUNK), jnp.float32),
            pltpu.VMEM((CHUNK,), jnp.int32),
            pltpu.VMEM((CHUNK,), jnp.int32),
            pltpu.VMEM((CHUNK,), jnp.float32),
            pltpu.SemaphoreType.DMA,
            pltpu.SemaphoreType.DMA,
            pltpu.SemaphoreType.DMA,
            pltpu.SemaphoreType.DMA,
            pltpu.SemaphoreType.DMA,
            pltpu.SemaphoreType.DMA,
        ],
    )
    def prop(x0_h, src_h, dst_h, val_h, out_h, agg, rowsA, rowsB,
             sid0, did0, val0, sid1, did1, val1, didxA, didxB, val1d,
             semGA, semGB, semSA, semSB, semI0, semI1):
        sid = lax.axis_index("s")
        crow0 = sid * NCH
        row0 = sid * ROWS_PT

        def idx_load(sc, sbuf, dbuf, vbuf, sem):
            r = crow0 + sc * SCH
            pltpu.make_async_copy(src_h.at[pl.ds(r, SCH)], sbuf, sem).start()
            pltpu.make_async_copy(dst_h.at[pl.ds(r, SCH)], dbuf, sem).start()
            pltpu.make_async_copy(val_h.at[pl.ds(r, SCH)], vbuf, sem).start()

        def idx_wait(sbuf, dbuf, vbuf, sem):
            pltpu.make_async_copy(src_h.at[pl.ds(0, SCH)], sbuf, sem).wait()
            pltpu.make_async_copy(dst_h.at[pl.ds(0, SCH)], dbuf, sem).wait()
            pltpu.make_async_copy(val_h.at[pl.ds(0, SCH)], vbuf, sem).wait()

        def g_start(sbuf, r, rows, sem):
            pltpu.make_async_copy(out_h.at[sbuf.at[r]], rows, sem).start()

        def g_wait(sbuf, r, rows, sem):
            pltpu.make_async_copy(out_h.at[sbuf.at[r]], rows, sem).wait()

        def sc_start(rows, dbuf, r, didx, sem):
            for d in range(CHUNK // 16):
                sl = pl.ds(d * 16, 16)
                didx[sl] = dbuf[r, sl]
            pltpu.make_async_copy(rows, agg.at[didx], sem).start(add=True)

        def sc_wait(sem):
            pltpu.make_async_copy(rowsA, agg.at[didxA], sem).wait()

        def scale(rows, vbuf, r):
            for d in range(CHUNK // 16):
                sl = pl.ds(d * 16, 16)
                val1d[sl] = vbuf[r, sl]

            @plsc.parallel_loop(0, CHUNK, 1, unroll=8)
            def _(e):
                vv = plsc.load_gather(val1d, [jnp.full((16,), e, jnp.int32)])
                for d in range(D // 16):
                    sl = pl.ds(d * 16, 16)
                    rows[e, sl] = rows[e, sl] * vv

        def pipeline(sbuf, dbuf, vbuf, reload_fn, not_first=None):
            if not_first is None:
                sc_wait(semSA)
            else:
                @pl.when(not_first)
                def _():
                    sc_wait(semSA)
            g_start(sbuf, 0, rowsA, semGA)
            reload_fn()
            for p in range(SCH // 2):
                ra, rb = 2 * p, 2 * p + 1
                if p == 0 and not_first is not None:
                    @pl.when(not_first)
                    def _():
                        sc_wait(semSB)
                else:
                    sc_wait(semSB)
                g_start(sbuf, rb, rowsB, semGB)
                g_wait(sbuf, ra, rowsA, semGA)
                scale(rowsA, vbuf, ra)
                sc_start(rowsA, dbuf, ra, didxA, semSA)
                g_wait(sbuf, rb, rowsB, semGB)
                scale(rowsB, vbuf, rb)
                sc_start(rowsB, dbuf, rb, didxB, semSB)
                if p < SCH // 2 - 1:
                    sc_wait(semSA)
                    g_start(sbuf, ra + 2, rowsA, semGA)

        def zrows(buf):
            def zb(i, c):
                for d in range(D // 16):
                    buf[i, pl.ds(d * 16, 16)] = jnp.zeros((16,), jnp.float32)
                return c
            lax.fori_loop(0, RCH, zb, 0, unroll=8)

        zrows(rowsB)
        for j in range(ROWS_PT // RCH):
            r = row0 + j * RCH
            pltpu.sync_copy(x0_h.at[pl.ds(r, RCH)], rowsA)
            pltpu.sync_copy(rowsA, out_h.at[pl.ds(r, RCH)])
            pltpu.sync_copy(rowsB, agg.at[pl.ds(r, RCH)])
        plsc.subcore_barrier()

        def iter_body(it, carry):
            idx_load(0, sid0, did0, val0, semI0)

            def sp_body(sp, c2):
                idx_wait(sid0, did0, val0, semI0)

                def reload1():
                    idx_load(2 * sp + 1, sid1, did1, val1, semI1)
                pipeline(sid0, did0, val0, reload1, not_first=sp > 0)

                idx_wait(sid1, did1, val1, semI1)

                def reload0():
                    @pl.when(sp < NSC // 2 - 1)
                    def _():
                        idx_load(2 * sp + 2, sid0, did0, val0, semI0)
                pipeline(sid1, did1, val1, reload0)
                return c2

            lax.fori_loop(0, NSC // 2, sp_body, 0)
            sc_wait(semSA)
            sc_wait(semSB)
            plsc.subcore_barrier()

            for j in range(ROWS_PT // RCH):
                r = row0 + j * RCH
                pltpu.sync_copy(agg.at[pl.ds(r, RCH)], rowsA)
                pltpu.sync_copy(x0_h.at[pl.ds(r, RCH)], rowsB)

                def mix(i, c3):
                    for d in range(D // 16):
                        sl = pl.ds(d * 16, 16)
                        a = rowsA[i, sl]
                        p0 = rowsB[i, sl]
                        rowsB[i, sl] = jnp.maximum(
                            a * (1.0 - ALPHA) + p0 * ALPHA, 0.0)
                    return c3
                lax.fori_loop(0, RCH, mix, 0, unroll=4)
                zrows(rowsA)
                pltpu.sync_copy(rowsA, agg.at[pl.ds(r, RCH)])
                pltpu.sync_copy(rowsB, out_h.at[pl.ds(r, RCH)])
            plsc.subcore_barrier()
            return carry

        lax.fori_loop(0, ITERS, iter_body, 0)

    return prop(x0, srcp, dstp, valp)


def kernel(x, W, edge_index, edge_vals):
    xp = jnp.pad(x, ((0, NP - N), (0, 0)))
    x0 = _matmul(xp, W)
    dst = edge_index[0]
    src = edge_index[1]
    spread = (jnp.arange(EPT - EPT_REAL, dtype=jnp.int32) * 37) % N
    srcp = _pad_edges(src, spread)
    dstp = _pad_edges(dst, spread)
    valp = _pad_edges(edge_vals, jnp.zeros((EPT - EPT_REAL,), jnp.float32))
    return _sc_propagate(x0, srcp, dstp, valp)[:N]
